# baseline (speedup 1.0000x reference)
import functools

import jax
import jax.numpy as jnp
from jax import lax
from jax.experimental import pallas as pl
from jax.experimental.pallas import tpu as pltpu

_BN = 16384


def _body(xt_ref, wt_ref, b_ref, a_ref, o_ref):
    xt = xt_ref[:, :]
    y = lax.dot_general(xt, wt_ref[:, :],
                        dimension_numbers=(((0,), (0,)), ((), ())),
                        preferred_element_type=jnp.float32) + b_ref[:, :]
    a = a_ref[0, 0]
    o_ref[:, :] = jnp.maximum(y, a * y)


@jax.jit
def kernel(last, W, b, prelu_a):
    n, idim = last.shape
    odim = W.shape[0]
    xt = last.T
    wt = W.T
    b2 = b.reshape(1, odim)
    a2 = jnp.asarray(prelu_a, jnp.float32).reshape(1, 1)
    grid = (n // _BN,)
    return pl.pallas_call(
        _body,
        grid=grid,
        in_specs=[
            pl.BlockSpec((idim, _BN), lambda i: (0, i)),
            pl.BlockSpec((idim, odim), lambda i: (0, 0)),
            pl.BlockSpec((1, odim), lambda i: (0, 0)),
            pl.BlockSpec((1, 1), lambda i: (0, 0)),
        ],
        out_specs=pl.BlockSpec((_BN, odim), lambda i: (i, 0)),
        out_shape=jax.ShapeDtypeStruct((n, odim), jnp.float32),
        compiler_params=pltpu.CompilerParams(
            allow_input_fusion=[True, False, False, False]),
    )(xt, wt, b2, a2)


# speedup vs baseline: 1.0213x; 1.0213x over previous
import functools

import jax
import jax.numpy as jnp
from jax import lax
from jax.experimental import pallas as pl
from jax.experimental.pallas import tpu as pltpu

_BN = 8192


def _body(xt_ref, wt_ref, b_ref, a_ref, o_ref):
    xt = xt_ref[:, :]
    y = lax.dot_general(xt, wt_ref[:, :],
                        dimension_numbers=(((0,), (0,)), ((), ())),
                        preferred_element_type=jnp.float32) + b_ref[:, :]
    a = a_ref[0, 0]
    o_ref[:, :] = jnp.maximum(y, a * y)


@jax.jit
def kernel(last, W, b, prelu_a):
    n, idim = last.shape
    odim = W.shape[0]
    xt = last.T
    wt = W.T
    b2 = b.reshape(1, odim)
    a2 = jnp.asarray(prelu_a, jnp.float32).reshape(1, 1)
    grid = (n // _BN,)
    return pl.pallas_call(
        _body,
        grid=grid,
        in_specs=[
            pl.BlockSpec((idim, _BN), lambda i: (0, i)),
            pl.BlockSpec((idim, odim), lambda i: (0, 0)),
            pl.BlockSpec((1, odim), lambda i: (0, 0)),
            pl.BlockSpec((1, 1), lambda i: (0, 0)),
        ],
        out_specs=pl.BlockSpec((_BN, odim), lambda i: (i, 0)),
        out_shape=jax.ShapeDtypeStruct((n, odim), jnp.float32),
        compiler_params=pltpu.CompilerParams(
            allow_input_fusion=[True, False, False, False]),
    )(xt, wt, b2, a2)
